# baseline (speedup 1.0000x reference)
import dataclasses
import functools

import jax
import jax.numpy as jnp
from jax import lax
from jax.experimental import pallas as pl
from jax.experimental.pallas import tpu as pltpu
from jax.experimental.pallas import tpu_sc as plsc

N = 10000
D = 128
E = 320000
NCLS = 40

NC = 2
NS = 16
NW = NC * NS
L = 16

EPAD = 327680
EPT = EPAD // NW
C = 80
NWIN = EPT // C
NB = 4
NPAD = 10240
RPT = NPAD // NS


def _encode(x, W_enc, b_enc, W1):
    def body(x_ref, we_ref, be_ref, w1_ref, o_ref):
        h = jnp.tanh(
            jnp.dot(x_ref[...], we_ref[...], preferred_element_type=jnp.float32)
            + be_ref[...]
        )
        o_ref[...] = jnp.dot(h, w1_ref[...], preferred_element_type=jnp.float32)

    return pl.pallas_call(
        body,
        out_shape=jax.ShapeDtypeStruct((N, D), jnp.float32),
    )(x, W_enc, b_enc.reshape(1, D), W1)


def _mid(parts, b, W):
    def body(p_ref, b_ref, w_ref, o_ref):
        h = jnp.tanh(p_ref[0, :N, :] + p_ref[1, :N, :] + b_ref[...])
        o_ref[...] = jnp.dot(h, w_ref[...], preferred_element_type=jnp.float32)

    return pl.pallas_call(
        body,
        out_shape=jax.ShapeDtypeStruct((N, D), jnp.float32),
    )(parts, b.reshape(1, D), W)


def _decode(parts, b2, W_dec, b_dec):
    def body(p_ref, b2_ref, wd_ref, bd_ref, o_ref):
        h = jnp.tanh(p_ref[0, :N, :] + p_ref[1, :N, :] + b2_ref[...])
        o_ref[...] = (
            jnp.dot(h, wd_ref[...], preferred_element_type=jnp.float32)
            + bd_ref[...]
        )

    return pl.pallas_call(
        body,
        out_shape=jax.ShapeDtypeStruct((N, NCLS), jnp.float32),
    )(parts, b2.reshape(1, D), W_dec, b_dec.reshape(1, NCLS))


def _sc_edge_pass(hw, src, dst, ew, zeros):
    mesh = plsc.VectorSubcoreMesh(core_axis_name="c", subcore_axis_name="s")
    cp = pltpu.CompilerParams()
    if "needs_layout_passes" in pltpu.CompilerParams.__dataclass_fields__:
        cp = dataclasses.replace(cp, needs_layout_passes=False)

    @functools.partial(
        pl.kernel,
        mesh=mesh,
        compiler_params=cp,
        out_type=jax.ShapeDtypeStruct((NC, NPAD, D), jnp.float32),
        scratch_types=(
            [pltpu.VMEM((C,), jnp.int32) for _ in range(NB)]
            + [pltpu.VMEM((C,), jnp.int32) for _ in range(NB)]
            + [pltpu.VMEM((C,), jnp.float32) for _ in range(NB)]
            + [pltpu.VMEM((C, D), jnp.float32) for _ in range(NB)]
            + [pltpu.VMEM_SHARED((NPAD, D), jnp.float32)]
            + [pltpu.SemaphoreType.DMA for _ in range(3 * NB)]
        ),
    )
    def k(hw_hbm, src_hbm, dst_hbm, ew_hbm, z_hbm, out_hbm, *refs):
        srcb = refs[0:NB]
        dstb = refs[NB:2 * NB]
        ewb = refs[2 * NB:3 * NB]
        rows = refs[3 * NB:4 * NB]
        acc_sh = refs[4 * NB]
        isem = refs[4 * NB + 1:4 * NB + 1 + NB]
        gsem = refs[4 * NB + 1 + NB:4 * NB + 1 + 2 * NB]
        ssem = refs[4 * NB + 1 + 2 * NB:4 * NB + 1 + 3 * NB]

        cid = lax.axis_index("c")
        sid = lax.axis_index("s")
        ebase = (cid * NS + sid) * EPT

        def idx_issue(wi, b):
            base = ebase + wi * C
            pltpu.async_copy(src_hbm.at[pl.ds(base, C)], srcb[b], isem[b])
            pltpu.async_copy(dst_hbm.at[pl.ds(base, C)], dstb[b], isem[b])
            pltpu.async_copy(ew_hbm.at[pl.ds(base, C)], ewb[b], isem[b])

        def idx_wait(wi, b):
            base = ebase + wi * C
            pltpu.make_async_copy(
                src_hbm.at[pl.ds(base, C)], srcb[b], isem[b]).wait()
            pltpu.make_async_copy(
                dst_hbm.at[pl.ds(base, C)], dstb[b], isem[b]).wait()
            pltpu.make_async_copy(
                ew_hbm.at[pl.ds(base, C)], ewb[b], isem[b]).wait()

        idx_issue(0, 0)
        idx_issue(1, 1)
        idx_issue(2, 2)
        idx_wait(0, 0)
        pltpu.async_copy(hw_hbm.at[srcb[0]], rows[0], gsem[0])
        idx_wait(1, 1)
        pltpu.async_copy(hw_hbm.at[srcb[1]], rows[1], gsem[1])

        pltpu.sync_copy(z_hbm.at[pl.ds(sid * RPT, RPT)],
                        acc_sh.at[pl.ds(sid * RPT, RPT)])
        plsc.subcore_barrier()

        @pl.loop(0, NWIN, step=NB)
        def _(w):
            for b in range(NB):
                wi = w + b
                b2 = (b + 2) % NB
                b3 = (b + 3) % NB

                @pl.when(wi + 3 < NWIN)
                def _():
                    idx_issue(wi + 3, b3)

                @pl.when(wi + 2 < NWIN)
                def _():
                    @pl.when(wi >= 2)
                    def _():
                        pltpu.make_async_copy(
                            rows[b2], acc_sh.at[dstb[b2]], ssem[b2]).wait()
                    idx_wait(wi + 2, b2)
                    pltpu.async_copy(hw_hbm.at[srcb[b2]], rows[b2], gsem[b2])

                pltpu.make_async_copy(
                    hw_hbm.at[srcb[b]], rows[b], gsem[b]).wait()

                @pl.loop(0, C)
                def _(r):
                    wvec = plsc.load_gather(
                        ewb[b], [jnp.full((L,), r, jnp.int32)])
                    for cc in range(D // L):
                        sl = pl.ds(cc * L, L)
                        rows[b][r, sl] = rows[b][r, sl] * wvec

                pltpu.async_copy(rows[b], acc_sh.at[dstb[b]], ssem[b],
                                 add=True)

        for b in range(NB):
            pltpu.make_async_copy(rows[b], acc_sh.at[dstb[b]], ssem[b]).wait()

        plsc.subcore_barrier()
        pltpu.sync_copy(acc_sh.at[pl.ds(sid * RPT, RPT)],
                        out_hbm.at[cid, pl.ds(sid * RPT, RPT)])

    return k(hw, src, dst, ew, zeros)


def kernel(x, edge_index, edge_weight, W_enc, b_enc, W1, b1, W2, b2, W_dec, b_dec):
    pad = EPAD - E
    src = jnp.concatenate(
        [edge_index[0].astype(jnp.int32), jnp.zeros((pad,), jnp.int32)])
    dst = jnp.concatenate(
        [edge_index[1].astype(jnp.int32), jnp.zeros((pad,), jnp.int32)])
    ew = jnp.concatenate(
        [edge_weight.astype(jnp.float32), jnp.zeros((pad,), jnp.float32)])
    zeros = jnp.zeros((NPAD, D), jnp.float32)

    hw1 = _encode(x, W_enc, b_enc, W1)
    p1 = _sc_edge_pass(hw1, src, dst, ew, zeros)
    hw2 = _mid(p1, b1, W2)
    p2 = _sc_edge_pass(hw2, src, dst, ew, zeros)
    return _decode(p2, b2, W_dec, b_dec)
